# baseline (speedup 1.0000x reference)
import functools

import jax
import jax.numpy as jnp
from jax import lax
from jax.experimental import pallas as pl
from jax.experimental.pallas import tpu as pltpu
from jax.experimental.pallas import tpu_sc as plsc

B = 16384
DMF = 16
DMLP = 32
U = 1000000

NC = 2
NS = 16
NW = NC * NS
CHUNK = 128
CH = B // (NW * CHUNK)
BPW = B // NW

TK = 8192


def _eye(n):
    return (lax.broadcasted_iota(jnp.int32, (n, n), 0)
            == lax.broadcasted_iota(jnp.int32, (n, n), 1)).astype(jnp.float32)


def _mxu_t(x, ident):
    return lax.dot_general(x, ident, (((0,), (0,)), ((), ())),
                           preferred_element_type=jnp.float32)


def _tr_body(umlp_ref, imlp_ref, umf_ref, imf_ref,
             o_umlp_ref, o_imlp_ref, o_umf_ref, o_imf_ref):
    i32 = _eye(DMLP)
    i16 = _eye(DMF)
    o_umlp_ref[...] = _mxu_t(umlp_ref[...], i32)
    o_imlp_ref[...] = _mxu_t(imlp_ref[...], i32)
    o_umf_ref[...] = _mxu_t(umf_ref[...], i16)
    o_imf_ref[...] = _mxu_t(imf_ref[...], i16)


_transpose_tables = pl.pallas_call(
    _tr_body,
    grid=(pl.cdiv(U, TK),),
    in_specs=[
        pl.BlockSpec((DMLP, TK), lambda i: (0, i)),
        pl.BlockSpec((DMLP, TK), lambda i: (0, i)),
        pl.BlockSpec((DMF, TK), lambda i: (0, i)),
        pl.BlockSpec((DMF, TK), lambda i: (0, i)),
    ],
    out_specs=[
        pl.BlockSpec((TK, DMLP), lambda i: (i, 0)),
        pl.BlockSpec((TK, DMLP), lambda i: (i, 0)),
        pl.BlockSpec((TK, DMF), lambda i: (i, 0)),
        pl.BlockSpec((TK, DMF), lambda i: (i, 0)),
    ],
    out_shape=[
        jax.ShapeDtypeStruct((U, DMLP), jnp.float32),
        jax.ShapeDtypeStruct((U, DMLP), jnp.float32),
        jax.ShapeDtypeStruct((U, DMF), jnp.float32),
        jax.ShapeDtypeStruct((U, DMF), jnp.float32),
    ],
)


@functools.lru_cache(maxsize=None)
def _build_sc_gather():
    mesh = plsc.VectorSubcoreMesh(core_axis_name="c", subcore_axis_name="s")

    @functools.partial(
        pl.kernel,
        out_type=(
            jax.ShapeDtypeStruct((B, DMLP), jnp.float32),
            jax.ShapeDtypeStruct((B, DMLP), jnp.float32),
            jax.ShapeDtypeStruct((B, DMF), jnp.float32),
            jax.ShapeDtypeStruct((B, DMF), jnp.float32),
        ),
        mesh=mesh,
        compiler_params=pltpu.CompilerParams(use_tc_tiling_on_sc=False),
        scratch_types=(
            pltpu.VMEM((BPW,), jnp.int32),
            pltpu.VMEM((BPW,), jnp.int32),
            pltpu.VMEM((BPW, DMLP), jnp.float32),
            pltpu.VMEM((BPW, DMLP), jnp.float32),
            pltpu.VMEM((BPW, DMF), jnp.float32),
            pltpu.VMEM((BPW, DMF), jnp.float32),
            pltpu.SemaphoreType.DMA,
        ),
    )
    def _sc_gather(uidx_hbm, iidx_hbm, t_umlp, t_imlp, t_umf, t_imf,
                   o_umlp, o_imlp, o_umf, o_imf,
                   uidx_v, iidx_v, b_umlp, b_imlp, b_umf, b_imf, sem):
        wid = lax.axis_index("s") * NC + lax.axis_index("c")
        base = wid * BPW
        pltpu.sync_copy(uidx_hbm.at[pl.ds(base, BPW)], uidx_v)
        pltpu.sync_copy(iidx_hbm.at[pl.ds(base, BPW)], iidx_v)
        cps = []
        for j in range(CH):
            sl = pl.ds(j * CHUNK, CHUNK)
            cps.append(pltpu.async_copy(t_umlp.at[uidx_v.at[sl]], b_umlp.at[sl], sem))
            cps.append(pltpu.async_copy(t_imlp.at[iidx_v.at[sl]], b_imlp.at[sl], sem))
            cps.append(pltpu.async_copy(t_umf.at[uidx_v.at[sl]], b_umf.at[sl], sem))
            cps.append(pltpu.async_copy(t_imf.at[iidx_v.at[sl]], b_imf.at[sl], sem))
        for cp in cps:
            cp.wait()
        osl = pl.ds(base, BPW)
        pltpu.sync_copy(b_umlp, o_umlp.at[osl])
        pltpu.sync_copy(b_imlp, o_imlp.at[osl])
        pltpu.sync_copy(b_umf, o_umf.at[osl])
        pltpu.sync_copy(b_imf, o_imf.at[osl])

    return _sc_gather


BT = 2048


def _leaky(x):
    return jnp.where(x >= 0, x, 0.01 * x)


def _tc_head(xu_ref, xi_ref, mu_ref, mi_ref, w0u_ref, w0i_ref, b0_ref,
             w1_ref, b1_ref, cwm_ref, cwf_ref, cb_ref, aw_ref, ab_ref,
             out_ref):
    hi = lax.Precision.HIGHEST
    xu = xu_ref[...].astype(jnp.float32)
    xi = xi_ref[...].astype(jnp.float32)
    h0 = (jnp.dot(xu, w0u_ref[...], precision=hi)
          + jnp.dot(xi, w0i_ref[...], precision=hi)
          + b0_ref[...])
    h0 = _leaky(h0)
    h1 = _leaky(jnp.dot(h0, w1_ref[...], precision=hi) + b1_ref[...])
    mf = (mu_ref[...].astype(jnp.float32)
         * mi_ref[...].astype(jnp.float32))
    v = _leaky(jnp.dot(h1, cwm_ref[...], precision=hi)
               + jnp.dot(mf, cwf_ref[...], precision=hi)
               + cb_ref[...])
    logit = jnp.sum(v * aw_ref[...], axis=1, keepdims=True) + ab_ref[...]
    out_ref[...] = 1.0 / (1.0 + jnp.exp(-logit))


def _full(shape):
    return pl.BlockSpec(shape, lambda i: (0, 0))


_mlp_head = pl.pallas_call(
    _tc_head,
    grid=(B // BT,),
    in_specs=[
        pl.BlockSpec((BT, DMLP), lambda i: (i, 0)),
        pl.BlockSpec((BT, DMLP), lambda i: (i, 0)),
        pl.BlockSpec((BT, DMF), lambda i: (i, 0)),
        pl.BlockSpec((BT, DMF), lambda i: (i, 0)),
        _full((DMLP, 128)),
        _full((DMLP, 128)),
        _full((1, 128)),
        _full((128, 64)),
        _full((1, 64)),
        _full((64, 64)),
        _full((DMF, 64)),
        _full((1, 64)),
        _full((1, 64)),
        _full((1, 1)),
    ],
    out_specs=pl.BlockSpec((BT, 1), lambda i: (i, 0)),
    out_shape=jax.ShapeDtypeStruct((B, 1), jnp.float32),
)


def kernel(user_indices, item_indices, emb_acc_mlp, emb_loc_mlp,
           emb_acc_mf, emb_loc_mf, fc0_w, fc0_b, fc1_w, fc1_b,
           common_w, common_b, aff_w, aff_b):
    uidx = user_indices.astype(jnp.int32)
    iidx = item_indices.astype(jnp.int32)
    r_umlp, r_imlp, r_umf, r_imf = _transpose_tables(
        emb_acc_mlp.T, emb_loc_mlp.T, emb_acc_mf.T, emb_loc_mf.T)
    u_mlp, i_mlp, u_mf, i_mf = _build_sc_gather()(
        uidx, iidx, r_umlp, r_imlp, r_umf, r_imf)

    w0 = fc0_w.T
    w0u, w0i = w0[:DMLP], w0[DMLP:]
    w1 = fc1_w.T
    cw = common_w.T
    cwm, cwf = cw[:64], cw[64:]
    return _mlp_head(
        u_mlp, i_mlp, u_mf, i_mf,
        w0u, w0i, fc0_b.reshape(1, 128),
        w1, fc1_b.reshape(1, 64),
        cwm, cwf, common_b.reshape(1, 64),
        aff_w, aff_b.reshape(1, 1))


# speedup vs baseline: 1.9883x; 1.9883x over previous
import functools

import jax
import jax.numpy as jnp
from jax import lax
from jax.experimental import pallas as pl
from jax.experimental.pallas import tpu as pltpu
from jax.experimental.pallas import tpu_sc as plsc

B = 16384
DMF = 16
DMLP = 32
U = 1000000
FW = 128

NC = 2
NS = 16
NW = NC * NS
CHUNK = 128
CH = B // (NW * CHUNK)
BPW = B // NW

C_UMLP = 0
C_IMLP = DMLP
C_UMF = 2 * DMLP
C_IMF = 2 * DMLP + DMF

TK = 8192


def _eye(n):
    return (lax.broadcasted_iota(jnp.int32, (n, n), 0)
            == lax.broadcasted_iota(jnp.int32, (n, n), 1)).astype(jnp.float32)


def _mxu_t(x, ident):
    return lax.dot_general(x, ident, (((0,), (0,)), ((), ())),
                           preferred_element_type=jnp.float32)


def _tr_body(umlp_ref, imlp_ref, umf_ref, imf_ref, out_ref):
    i32 = _eye(DMLP)
    i16 = _eye(DMF)
    out_ref[:, C_UMLP:C_UMLP + DMLP] = _mxu_t(umlp_ref[...], i32)
    out_ref[:, C_IMLP:C_IMLP + DMLP] = _mxu_t(imlp_ref[...], i32)
    out_ref[:, C_UMF:C_UMF + DMF] = _mxu_t(umf_ref[...], i16)
    out_ref[:, C_IMF:C_IMF + DMF] = _mxu_t(imf_ref[...], i16)


_fuse_tables = pl.pallas_call(
    _tr_body,
    grid=(pl.cdiv(U, TK),),
    in_specs=[
        pl.BlockSpec((DMLP, TK), lambda i: (0, i)),
        pl.BlockSpec((DMLP, TK), lambda i: (0, i)),
        pl.BlockSpec((DMF, TK), lambda i: (0, i)),
        pl.BlockSpec((DMF, TK), lambda i: (0, i)),
    ],
    out_specs=pl.BlockSpec((TK, FW), lambda i: (i, 0)),
    out_shape=jax.ShapeDtypeStruct((U, FW), jnp.float32),
)


@functools.lru_cache(maxsize=None)
def _build_sc_gather():
    mesh = plsc.VectorSubcoreMesh(core_axis_name="c", subcore_axis_name="s")

    @functools.partial(
        pl.kernel,
        out_type=(
            jax.ShapeDtypeStruct((B, DMLP), jnp.float32),
            jax.ShapeDtypeStruct((B, DMLP), jnp.float32),
            jax.ShapeDtypeStruct((B, DMF), jnp.float32),
            jax.ShapeDtypeStruct((B, DMF), jnp.float32),
        ),
        mesh=mesh,
        compiler_params=pltpu.CompilerParams(use_tc_tiling_on_sc=False),
        scratch_types=(
            pltpu.VMEM((BPW,), jnp.int32),
            pltpu.VMEM((BPW,), jnp.int32),
            pltpu.VMEM((BPW, FW), jnp.float32),
            pltpu.SemaphoreType.DMA,
        ),
    )
    def _sc_gather(uidx_hbm, iidx_hbm, table,
                   o_umlp, o_imlp, o_umf, o_imf,
                   uidx_v, iidx_v, rows_v, sem):
        wid = lax.axis_index("s") * NC + lax.axis_index("c")
        base = wid * BPW
        osl = pl.ds(base, BPW)
        pltpu.sync_copy(uidx_hbm.at[osl], uidx_v)
        pltpu.sync_copy(iidx_hbm.at[osl], iidx_v)

        cps = []
        for j in range(CH):
            sl = pl.ds(j * CHUNK, CHUNK)
            cps.append(pltpu.async_copy(table.at[uidx_v.at[sl]], rows_v.at[sl], sem))
        for cp in cps:
            cp.wait()
        pltpu.sync_copy(rows_v.at[:, pl.ds(C_UMLP, DMLP)], o_umlp.at[osl])
        pltpu.sync_copy(rows_v.at[:, pl.ds(C_UMF, DMF)], o_umf.at[osl])

        cps = []
        for j in range(CH):
            sl = pl.ds(j * CHUNK, CHUNK)
            cps.append(pltpu.async_copy(table.at[iidx_v.at[sl]], rows_v.at[sl], sem))
        for cp in cps:
            cp.wait()
        pltpu.sync_copy(rows_v.at[:, pl.ds(C_IMLP, DMLP)], o_imlp.at[osl])
        pltpu.sync_copy(rows_v.at[:, pl.ds(C_IMF, DMF)], o_imf.at[osl])

    return _sc_gather


BT = 2048


def _leaky(x):
    return jnp.where(x >= 0, x, 0.01 * x)


def _tc_head(xu_ref, xi_ref, mu_ref, mi_ref, w0u_ref, w0i_ref, b0_ref,
             w1_ref, b1_ref, cwm_ref, cwf_ref, cb_ref, aw_ref, ab_ref,
             out_ref):
    hi = lax.Precision.HIGHEST
    h0 = (jnp.dot(xu_ref[...], w0u_ref[...], precision=hi)
          + jnp.dot(xi_ref[...], w0i_ref[...], precision=hi)
          + b0_ref[...])
    h0 = _leaky(h0)
    h1 = _leaky(jnp.dot(h0, w1_ref[...], precision=hi) + b1_ref[...])
    mf = mu_ref[...] * mi_ref[...]
    v = _leaky(jnp.dot(h1, cwm_ref[...], precision=hi)
               + jnp.dot(mf, cwf_ref[...], precision=hi)
               + cb_ref[...])
    logit = jnp.sum(v * aw_ref[...], axis=1, keepdims=True) + ab_ref[...]
    out_ref[...] = 1.0 / (1.0 + jnp.exp(-logit))


def _full(shape):
    return pl.BlockSpec(shape, lambda i: (0, 0))


_mlp_head = pl.pallas_call(
    _tc_head,
    grid=(B // BT,),
    in_specs=[
        pl.BlockSpec((BT, DMLP), lambda i: (i, 0)),
        pl.BlockSpec((BT, DMLP), lambda i: (i, 0)),
        pl.BlockSpec((BT, DMF), lambda i: (i, 0)),
        pl.BlockSpec((BT, DMF), lambda i: (i, 0)),
        _full((DMLP, 128)),
        _full((DMLP, 128)),
        _full((1, 128)),
        _full((128, 64)),
        _full((1, 64)),
        _full((64, 64)),
        _full((DMF, 64)),
        _full((1, 64)),
        _full((1, 64)),
        _full((1, 1)),
    ],
    out_specs=pl.BlockSpec((BT, 1), lambda i: (i, 0)),
    out_shape=jax.ShapeDtypeStruct((B, 1), jnp.float32),
)


def kernel(user_indices, item_indices, emb_acc_mlp, emb_loc_mlp,
           emb_acc_mf, emb_loc_mf, fc0_w, fc0_b, fc1_w, fc1_b,
           common_w, common_b, aff_w, aff_b):
    uidx = user_indices.astype(jnp.int32)
    iidx = item_indices.astype(jnp.int32)
    table = _fuse_tables(
        emb_acc_mlp.T, emb_loc_mlp.T, emb_acc_mf.T, emb_loc_mf.T)
    u_mlp, i_mlp, u_mf, i_mf = _build_sc_gather()(uidx, iidx, table)

    w0 = fc0_w.T
    w0u, w0i = w0[:DMLP], w0[DMLP:]
    w1 = fc1_w.T
    cw = common_w.T
    cwm, cwf = cw[:64], cw[64:]
    return _mlp_head(
        u_mlp, i_mlp, u_mf, i_mf,
        w0u, w0i, fc0_b.reshape(1, 128),
        w1, fc1_b.reshape(1, 64),
        cwm, cwf, common_b.reshape(1, 64),
        aff_w, aff_b.reshape(1, 1))


# speedup vs baseline: 5.5170x; 2.7748x over previous
import functools

import jax
import jax.numpy as jnp
from jax import lax
from jax.experimental import pallas as pl
from jax.experimental.pallas import tpu as pltpu
from jax.experimental.pallas import tpu_sc as plsc

B = 16384
DMF = 16
DMLP = 32
U = 1000000
FW = 128
DSTK = 2 * DMLP + 2 * DMF

NC = 2
NS = 16
NW = NC * NS
CHUNK = 128
CH = B // (NW * CHUNK)
BPW = B // NW

C_UMLP = 0
C_IMLP = DMLP
C_UMF = 2 * DMLP
C_IMF = 2 * DMLP + DMF

TK = 16384


def _tr_body(umlp_ref, imlp_ref, umf_ref, imf_ref, out_ref):
    stack = jnp.concatenate(
        [umlp_ref[...], imlp_ref[...], umf_ref[...], imf_ref[...]], axis=0)
    emb = (lax.broadcasted_iota(jnp.int32, (DSTK, FW), 0)
           == lax.broadcasted_iota(jnp.int32, (DSTK, FW), 1)).astype(jnp.float32)
    out_ref[...] = lax.dot_general(stack, emb, (((0,), (0,)), ((), ())),
                                   preferred_element_type=jnp.float32)


_fuse_tables = pl.pallas_call(
    _tr_body,
    grid=(pl.cdiv(U, TK),),
    in_specs=[
        pl.BlockSpec((DMLP, TK), lambda i: (0, i)),
        pl.BlockSpec((DMLP, TK), lambda i: (0, i)),
        pl.BlockSpec((DMF, TK), lambda i: (0, i)),
        pl.BlockSpec((DMF, TK), lambda i: (0, i)),
    ],
    out_specs=pl.BlockSpec((TK, FW), lambda i: (i, 0)),
    out_shape=jax.ShapeDtypeStruct((U, FW), jnp.float32),
)


@functools.lru_cache(maxsize=None)
def _build_sc_gather():
    mesh = plsc.VectorSubcoreMesh(core_axis_name="c", subcore_axis_name="s")

    @functools.partial(
        pl.kernel,
        out_type=(
            jax.ShapeDtypeStruct((B, FW), jnp.float32),
            jax.ShapeDtypeStruct((B, FW), jnp.float32),
        ),
        mesh=mesh,
        compiler_params=pltpu.CompilerParams(use_tc_tiling_on_sc=False),
        scratch_types=(
            pltpu.VMEM((BPW,), jnp.int32),
            pltpu.VMEM((BPW,), jnp.int32),
            pltpu.VMEM((BPW, FW), jnp.float32),
            pltpu.SemaphoreType.DMA,
        ),
    )
    def _sc_gather(uidx_hbm, iidx_hbm, table, o_u, o_i,
                   uidx_v, iidx_v, rows_v, sem):
        wid = lax.axis_index("s") * NC + lax.axis_index("c")
        base = wid * BPW
        osl = pl.ds(base, BPW)
        pltpu.sync_copy(uidx_hbm.at[osl], uidx_v)
        pltpu.sync_copy(iidx_hbm.at[osl], iidx_v)

        cps = []
        for j in range(CH):
            sl = pl.ds(j * CHUNK, CHUNK)
            cps.append(pltpu.async_copy(table.at[uidx_v.at[sl]], rows_v.at[sl], sem))
        for cp in cps:
            cp.wait()
        pltpu.sync_copy(rows_v, o_u.at[osl])

        cps = []
        for j in range(CH):
            sl = pl.ds(j * CHUNK, CHUNK)
            cps.append(pltpu.async_copy(table.at[iidx_v.at[sl]], rows_v.at[sl], sem))
        for cp in cps:
            cp.wait()
        pltpu.sync_copy(rows_v, o_i.at[osl])

    return _sc_gather


BT = 2048


def _leaky(x):
    return jnp.where(x >= 0, x, 0.01 * x)


def _tc_head(u_ref, i_ref, w0u_ref, w0i_ref, b0_ref,
             w1_ref, b1_ref, cwm_ref, cwf_ref, cb_ref, aw_ref, ab_ref,
             out_ref):
    hi = lax.Precision.HIGHEST
    u = u_ref[...]
    i = i_ref[...]
    xu = u[:, C_UMLP:C_UMLP + DMLP]
    xi = i[:, C_IMLP:C_IMLP + DMLP]
    mf = u[:, C_UMF:C_UMF + DMF] * i[:, C_IMF:C_IMF + DMF]
    h0 = (jnp.dot(xu, w0u_ref[...], precision=hi)
          + jnp.dot(xi, w0i_ref[...], precision=hi)
          + b0_ref[...])
    h0 = _leaky(h0)
    h1 = _leaky(jnp.dot(h0, w1_ref[...], precision=hi) + b1_ref[...])
    v = _leaky(jnp.dot(h1, cwm_ref[...], precision=hi)
               + jnp.dot(mf, cwf_ref[...], precision=hi)
               + cb_ref[...])
    logit = jnp.sum(v * aw_ref[...], axis=1, keepdims=True) + ab_ref[...]
    out_ref[...] = 1.0 / (1.0 + jnp.exp(-logit))


def _full(shape):
    return pl.BlockSpec(shape, lambda i: (0, 0))


_mlp_head = pl.pallas_call(
    _tc_head,
    grid=(B // BT,),
    in_specs=[
        pl.BlockSpec((BT, FW), lambda i: (i, 0)),
        pl.BlockSpec((BT, FW), lambda i: (i, 0)),
        _full((DMLP, 128)),
        _full((DMLP, 128)),
        _full((1, 128)),
        _full((128, 64)),
        _full((1, 64)),
        _full((64, 64)),
        _full((DMF, 64)),
        _full((1, 64)),
        _full((1, 64)),
        _full((1, 1)),
    ],
    out_specs=pl.BlockSpec((BT, 1), lambda i: (i, 0)),
    out_shape=jax.ShapeDtypeStruct((B, 1), jnp.float32),
)


def kernel(user_indices, item_indices, emb_acc_mlp, emb_loc_mlp,
           emb_acc_mf, emb_loc_mf, fc0_w, fc0_b, fc1_w, fc1_b,
           common_w, common_b, aff_w, aff_b):
    uidx = user_indices.astype(jnp.int32)
    iidx = item_indices.astype(jnp.int32)
    table = _fuse_tables(
        emb_acc_mlp.T, emb_loc_mlp.T, emb_acc_mf.T, emb_loc_mf.T)
    g_u, g_i = _build_sc_gather()(uidx, iidx, table)

    w0 = fc0_w.T
    w0u, w0i = w0[:DMLP], w0[DMLP:]
    w1 = fc1_w.T
    cw = common_w.T
    cwm, cwf = cw[:64], cw[64:]
    return _mlp_head(
        g_u, g_i,
        w0u, w0i, fc0_b.reshape(1, 128),
        w1, fc1_b.reshape(1, 64),
        cwm, cwf, common_b.reshape(1, 64),
        aff_w, aff_b.reshape(1, 1))


# speedup vs baseline: 6.3861x; 1.1575x over previous
import functools

import jax
import jax.numpy as jnp
from jax import lax
from jax.experimental import pallas as pl
from jax.experimental.pallas import tpu as pltpu
from jax.experimental.pallas import tpu_sc as plsc

B = 16384
DMF = 16
DMLP = 32
U = 1000000
FW = 128
DSTK = 2 * DMLP + 2 * DMF

NC = 2
NS = 16
NW = NC * NS
CHUNK = 128
CH = B // (NW * CHUNK)
BPW = B // NW

C_UMLP = 0
C_IMLP = DMLP
C_UMF = 2 * DMLP
C_IMF = 2 * DMLP + DMF

TK = 32768


def _tr_body(umlp_ref, imlp_ref, umf_ref, imf_ref, out_ref):
    stack = jnp.concatenate(
        [umlp_ref[...], imlp_ref[...], umf_ref[...], imf_ref[...]], axis=0)
    emb = (lax.broadcasted_iota(jnp.int32, (DSTK, FW), 0)
           == lax.broadcasted_iota(jnp.int32, (DSTK, FW), 1)).astype(jnp.float32)
    out_ref[...] = lax.dot_general(stack, emb, (((0,), (0,)), ((), ())),
                                   preferred_element_type=jnp.float32)


_fuse_tables = pl.pallas_call(
    _tr_body,
    grid=(pl.cdiv(U, TK),),
    in_specs=[
        pl.BlockSpec((DMLP, TK), lambda i: (0, i)),
        pl.BlockSpec((DMLP, TK), lambda i: (0, i)),
        pl.BlockSpec((DMF, TK), lambda i: (0, i)),
        pl.BlockSpec((DMF, TK), lambda i: (0, i)),
    ],
    out_specs=pl.BlockSpec((TK, FW), lambda i: (i, 0)),
    out_shape=jax.ShapeDtypeStruct((U, FW), jnp.float32),
)


@functools.lru_cache(maxsize=None)
def _build_sc_gather():
    mesh = plsc.VectorSubcoreMesh(core_axis_name="c", subcore_axis_name="s")

    @functools.partial(
        pl.kernel,
        out_type=(
            jax.ShapeDtypeStruct((B, FW), jnp.float32),
            jax.ShapeDtypeStruct((B, FW), jnp.float32),
        ),
        mesh=mesh,
        compiler_params=pltpu.CompilerParams(use_tc_tiling_on_sc=False),
        scratch_types=(
            pltpu.VMEM((BPW,), jnp.int32),
            pltpu.VMEM((BPW,), jnp.int32),
            pltpu.VMEM((BPW, FW), jnp.float32),
            pltpu.SemaphoreType.DMA,
        ),
    )
    def _sc_gather(uidx_hbm, iidx_hbm, table, o_u, o_i,
                   uidx_v, iidx_v, rows_v, sem):
        wid = lax.axis_index("s") * NC + lax.axis_index("c")
        base = wid * BPW
        osl = pl.ds(base, BPW)
        pltpu.sync_copy(uidx_hbm.at[osl], uidx_v)
        pltpu.sync_copy(iidx_hbm.at[osl], iidx_v)

        cps = []
        for j in range(CH):
            sl = pl.ds(j * CHUNK, CHUNK)
            cps.append(pltpu.async_copy(table.at[uidx_v.at[sl]], rows_v.at[sl], sem))
        for cp in cps:
            cp.wait()
        pltpu.sync_copy(rows_v, o_u.at[osl])

        cps = []
        for j in range(CH):
            sl = pl.ds(j * CHUNK, CHUNK)
            cps.append(pltpu.async_copy(table.at[iidx_v.at[sl]], rows_v.at[sl], sem))
        for cp in cps:
            cp.wait()
        pltpu.sync_copy(rows_v, o_i.at[osl])

    return _sc_gather


BT = 2048


def _leaky(x):
    return jnp.where(x >= 0, x, 0.01 * x)


def _tc_head(u_ref, i_ref, w0u_ref, w0i_ref, b0_ref,
             w1_ref, b1_ref, cwm_ref, cwf_ref, cb_ref, aw_ref, ab_ref,
             out_ref):
    u = u_ref[...]
    i = i_ref[...]
    xu = u[:, C_UMLP:C_UMLP + DMLP]
    xi = i[:, C_IMLP:C_IMLP + DMLP]
    mf = u[:, C_UMF:C_UMF + DMF] * i[:, C_IMF:C_IMF + DMF]
    h0 = (jnp.dot(xu, w0u_ref[...])
          + jnp.dot(xi, w0i_ref[...])
          + b0_ref[...])
    h0 = _leaky(h0)
    h1 = _leaky(jnp.dot(h0, w1_ref[...]) + b1_ref[...])
    v = _leaky(jnp.dot(h1, cwm_ref[...])
               + jnp.dot(mf, cwf_ref[...])
               + cb_ref[...])
    logit = jnp.sum(v * aw_ref[...], axis=1, keepdims=True) + ab_ref[...]
    out_ref[...] = 1.0 / (1.0 + jnp.exp(-logit))


def _full(shape):
    return pl.BlockSpec(shape, lambda i: (0, 0))


_mlp_head = pl.pallas_call(
    _tc_head,
    grid=(B // BT,),
    in_specs=[
        pl.BlockSpec((BT, FW), lambda i: (i, 0)),
        pl.BlockSpec((BT, FW), lambda i: (i, 0)),
        _full((DMLP, 128)),
        _full((DMLP, 128)),
        _full((1, 128)),
        _full((128, 64)),
        _full((1, 64)),
        _full((64, 64)),
        _full((DMF, 64)),
        _full((1, 64)),
        _full((1, 64)),
        _full((1, 1)),
    ],
    out_specs=pl.BlockSpec((BT, 1), lambda i: (i, 0)),
    out_shape=jax.ShapeDtypeStruct((B, 1), jnp.float32),
)


def kernel(user_indices, item_indices, emb_acc_mlp, emb_loc_mlp,
           emb_acc_mf, emb_loc_mf, fc0_w, fc0_b, fc1_w, fc1_b,
           common_w, common_b, aff_w, aff_b):
    uidx = user_indices.astype(jnp.int32)
    iidx = item_indices.astype(jnp.int32)
    table = _fuse_tables(
        emb_acc_mlp.T, emb_loc_mlp.T, emb_acc_mf.T, emb_loc_mf.T)
    g_u, g_i = _build_sc_gather()(uidx, iidx, table)

    w0 = fc0_w.T
    w0u, w0i = w0[:DMLP], w0[DMLP:]
    w1 = fc1_w.T
    cw = common_w.T
    cwm, cwf = cw[:64], cw[64:]
    return _mlp_head(
        g_u, g_i,
        w0u, w0i, fc0_b.reshape(1, 128),
        w1, fc1_b.reshape(1, 64),
        cwm, cwf, common_b.reshape(1, 64),
        aff_w, aff_b.reshape(1, 1))


# speedup vs baseline: 6.3892x; 1.0005x over previous
import functools

import jax
import jax.numpy as jnp
from jax import lax
from jax.experimental import pallas as pl
from jax.experimental.pallas import tpu as pltpu
from jax.experimental.pallas import tpu_sc as plsc

B = 16384
DMF = 16
DMLP = 32
U = 1000000
FW = 128
DSTK = 2 * DMLP + 2 * DMF

NC = 2
NS = 16
NW = NC * NS
CHUNK = 128
CH = B // (NW * CHUNK)
BPW = B // NW

C_UMLP = 0
C_IMLP = DMLP
C_UMF = 2 * DMLP
C_IMF = 2 * DMLP + DMF

TK = 32768


def _tr_body(umlp_ref, imlp_ref, umf_ref, imf_ref, out_ref):
    stack = jnp.concatenate(
        [umlp_ref[...], imlp_ref[...], umf_ref[...], imf_ref[...]], axis=0)
    emb = (lax.broadcasted_iota(jnp.int32, (DSTK, FW), 0)
           == lax.broadcasted_iota(jnp.int32, (DSTK, FW), 1)).astype(jnp.float32)
    out_ref[...] = lax.dot_general(stack, emb, (((0,), (0,)), ((), ())),
                                   preferred_element_type=jnp.float32)


_fuse_tables = pl.pallas_call(
    _tr_body,
    grid=(pl.cdiv(U, TK),),
    in_specs=[
        pl.BlockSpec((DMLP, TK), lambda i: (0, i)),
        pl.BlockSpec((DMLP, TK), lambda i: (0, i)),
        pl.BlockSpec((DMF, TK), lambda i: (0, i)),
        pl.BlockSpec((DMF, TK), lambda i: (0, i)),
    ],
    out_specs=pl.BlockSpec((TK, FW), lambda i: (i, 0)),
    out_shape=jax.ShapeDtypeStruct((U, FW), jnp.float32),
)


@functools.lru_cache(maxsize=None)
def _build_sc_gather():
    mesh = plsc.VectorSubcoreMesh(core_axis_name="c", subcore_axis_name="s")

    @functools.partial(
        pl.kernel,
        out_type=(
            jax.ShapeDtypeStruct((B, FW), jnp.float32),
            jax.ShapeDtypeStruct((B, FW), jnp.float32),
        ),
        mesh=mesh,
        compiler_params=pltpu.CompilerParams(use_tc_tiling_on_sc=False),
        scratch_types=(
            pltpu.VMEM((BPW,), jnp.int32),
            pltpu.VMEM((BPW,), jnp.int32),
            pltpu.VMEM((CH, CHUNK, FW), jnp.float32),
            pltpu.SemaphoreType.DMA,
            pltpu.SemaphoreType.DMA,
            pltpu.SemaphoreType.DMA,
            pltpu.SemaphoreType.DMA,
        ),
    )
    def _sc_gather(uidx_hbm, iidx_hbm, table, o_u, o_i,
                   uidx_v, iidx_v, rows_v, sem0, sem1, sem2, sem3):
        sems = (sem0, sem1, sem2, sem3)
        wid = lax.axis_index("s") * NC + lax.axis_index("c")
        base = wid * BPW
        osl = pl.ds(base, BPW)
        pltpu.sync_copy(uidx_hbm.at[osl], uidx_v)
        pltpu.sync_copy(iidx_hbm.at[osl], iidx_v)

        ucps = []
        for j in range(CH):
            sl = pl.ds(j * CHUNK, CHUNK)
            ucps.append(pltpu.async_copy(table.at[uidx_v.at[sl]], rows_v.at[j], sems[j]))
        icps = []
        for j in range(CH):
            sl = pl.ds(j * CHUNK, CHUNK)
            ucps[j].wait()
            pltpu.sync_copy(rows_v.at[j], o_u.at[pl.ds(base + j * CHUNK, CHUNK)])
            icps.append(pltpu.async_copy(table.at[iidx_v.at[sl]], rows_v.at[j], sems[j]))
        for j in range(CH):
            icps[j].wait()
            pltpu.sync_copy(rows_v.at[j], o_i.at[pl.ds(base + j * CHUNK, CHUNK)])

    return _sc_gather


BT = 2048


def _leaky(x):
    return jnp.where(x >= 0, x, 0.01 * x)


def _tc_head(u_ref, i_ref, w0u_ref, w0i_ref, b0_ref,
             w1_ref, b1_ref, cwm_ref, cwf_ref, cb_ref, aw_ref, ab_ref,
             out_ref):
    u = u_ref[...]
    i = i_ref[...]
    xu = u[:, C_UMLP:C_UMLP + DMLP]
    xi = i[:, C_IMLP:C_IMLP + DMLP]
    mf = u[:, C_UMF:C_UMF + DMF] * i[:, C_IMF:C_IMF + DMF]
    h0 = (jnp.dot(xu, w0u_ref[...])
          + jnp.dot(xi, w0i_ref[...])
          + b0_ref[...])
    h0 = _leaky(h0)
    h1 = _leaky(jnp.dot(h0, w1_ref[...]) + b1_ref[...])
    v = _leaky(jnp.dot(h1, cwm_ref[...])
               + jnp.dot(mf, cwf_ref[...])
               + cb_ref[...])
    logit = jnp.sum(v * aw_ref[...], axis=1, keepdims=True) + ab_ref[...]
    out_ref[...] = 1.0 / (1.0 + jnp.exp(-logit))


def _full(shape):
    return pl.BlockSpec(shape, lambda i: (0, 0))


_mlp_head = pl.pallas_call(
    _tc_head,
    grid=(B // BT,),
    in_specs=[
        pl.BlockSpec((BT, FW), lambda i: (i, 0)),
        pl.BlockSpec((BT, FW), lambda i: (i, 0)),
        _full((DMLP, 128)),
        _full((DMLP, 128)),
        _full((1, 128)),
        _full((128, 64)),
        _full((1, 64)),
        _full((64, 64)),
        _full((DMF, 64)),
        _full((1, 64)),
        _full((1, 64)),
        _full((1, 1)),
    ],
    out_specs=pl.BlockSpec((BT, 1), lambda i: (i, 0)),
    out_shape=jax.ShapeDtypeStruct((B, 1), jnp.float32),
)


def kernel(user_indices, item_indices, emb_acc_mlp, emb_loc_mlp,
           emb_acc_mf, emb_loc_mf, fc0_w, fc0_b, fc1_w, fc1_b,
           common_w, common_b, aff_w, aff_b):
    uidx = user_indices.astype(jnp.int32)
    iidx = item_indices.astype(jnp.int32)
    table = _fuse_tables(
        emb_acc_mlp.T, emb_loc_mlp.T, emb_acc_mf.T, emb_loc_mf.T)
    g_u, g_i = _build_sc_gather()(uidx, iidx, table)

    w0 = fc0_w.T
    w0u, w0i = w0[:DMLP], w0[DMLP:]
    w1 = fc1_w.T
    cw = common_w.T
    cwm, cwf = cw[:64], cw[64:]
    return _mlp_head(
        g_u, g_i,
        w0u, w0i, fc0_b.reshape(1, 128),
        w1, fc1_b.reshape(1, 64),
        cwm, cwf, common_b.reshape(1, 64),
        aff_w, aff_b.reshape(1, 1))
